# baseline (speedup 1.0000x reference)
import jax
import jax.numpy as jnp
from jax.experimental import pallas as pl

_BT = 1024


def _embed_kernel(x_ref, w_ref, b_ref, o_ref):
    o_ref[:] = (
        jnp.dot(x_ref[:], w_ref[:], preferred_element_type=jnp.float32)
        + b_ref[:]
    )


@jax.jit
def kernel(x, W0, b0, W1, b1, W2, b2, W3, b3):
    G, H = W0.shape
    n = 4
    D = G * n
    O = H * n
    Wb = jnp.zeros((D, O), x.dtype)
    for i, W in enumerate((W0, W1, W2, W3)):
        Wb = jax.lax.dynamic_update_slice(Wb, W, (i * G, i * H))
    bb = jnp.concatenate([b0, b1, b2, b3]).reshape(1, O)

    B = x.shape[0]
    return pl.pallas_call(
        _embed_kernel,
        grid=(B // _BT,),
        in_specs=[
            pl.BlockSpec((_BT, D), lambda i: (i, 0)),
            pl.BlockSpec((D, O), lambda i: (0, 0)),
            pl.BlockSpec((1, O), lambda i: (0, 0)),
        ],
        out_specs=pl.BlockSpec((_BT, O), lambda i: (i, 0)),
        out_shape=jax.ShapeDtypeStruct((B, O), x.dtype),
    )(x, Wb, bb)


# speedup vs baseline: 1.3318x; 1.3318x over previous
import jax
import jax.numpy as jnp
from jax.experimental import pallas as pl

_BT = 8192


def _embed_kernel(x_ref, w_ref, b_ref, o_ref):
    o_ref[:] = (
        jnp.dot(x_ref[:], w_ref[:], preferred_element_type=jnp.float32)
        + b_ref[:]
    )


@jax.jit
def kernel(x, W0, b0, W1, b1, W2, b2, W3, b3):
    G, H = W0.shape
    n = 4
    D = G * n
    O = H * n
    Wb = jnp.zeros((D, O), x.dtype)
    for i, W in enumerate((W0, W1, W2, W3)):
        Wb = jax.lax.dynamic_update_slice(Wb, W, (i * G, i * H))
    bb = jnp.concatenate([b0, b1, b2, b3]).reshape(1, O)

    B = x.shape[0]
    return pl.pallas_call(
        _embed_kernel,
        grid=(B // _BT,),
        in_specs=[
            pl.BlockSpec((_BT, D), lambda i: (i, 0)),
            pl.BlockSpec((D, O), lambda i: (0, 0)),
            pl.BlockSpec((1, O), lambda i: (0, 0)),
        ],
        out_specs=pl.BlockSpec((_BT, O), lambda i: (i, 0)),
        out_shape=jax.ShapeDtypeStruct((B, O), x.dtype),
    )(x, Wb, bb)
